# baseline (speedup 1.0000x reference)
import jax
import jax.numpy as jnp
from jax.experimental import pallas as pl

N_VEH = 6
N_PED = 2
LOGIT_DIM = 6

_B, _N, _D = 32, 256, 1024
_H = _D // 2
_T = _B * _N
_BLK = 1024
_HALFB = _T // (2 * _BLK)

_SQRT_HALF = 0.7071067811865476


def _head_pair(x, t, w1v, b1v, w2v, b2v, w1p, b1p, w2p, b2p):
    mv = t == 0
    mp = t == 1
    gv = jnp.dot(x, w1v, preferred_element_type=jnp.float32) + b1v
    hv = 0.5 * gv * (1.0 + jax.lax.erf(gv * _SQRT_HALF))
    gp = jnp.dot(x, w1p, preferred_element_type=jnp.float32) + b1p
    hp = 0.5 * gp * (1.0 + jax.lax.erf(gp * _SQRT_HALF))
    hv = hv * mv.astype(jnp.float32)
    hp = hp * mp.astype(jnp.float32)
    out = (jnp.dot(hv, w2v, preferred_element_type=jnp.float32)
           + jnp.dot(hp, w2p, preferred_element_type=jnp.float32))
    out = out + jnp.where(mv, b2v, 0.0) + jnp.where(mp, b2p, 0.0)
    return out, mv, mp


def _body(xa_ref, xb_ref, ta_ref, tb_ref, w1v_ref, b1v_ref, w2v_ref, b2v_ref,
          w1p_ref, b1p_ref, w2p_ref, b2p_ref,
          outa_ref, outb_ref, mva_ref, mvb_ref, mpa_ref, mpb_ref):
    args = (w1v_ref[...], b1v_ref[...], w2v_ref[...], b2v_ref[...],
            w1p_ref[...], b1p_ref[...], w2p_ref[...], b2p_ref[...])
    out, mv, mp = _head_pair(xa_ref[...], ta_ref[...], *args)
    outa_ref[...] = out
    mva_ref[...] = mv
    mpa_ref[...] = mp
    out, mv, mp = _head_pair(xb_ref[...], tb_ref[...], *args)
    outb_ref[...] = out
    mvb_ref[...] = mv
    mpb_ref[...] = mp


def kernel(repr3, agent_type_ids, W1v, b1v, W2v, b2v, W1p, b1p, W2p, b2p):
    x = repr3.reshape(_T, _D)
    t = agent_type_ids.reshape(_T, 1)

    w2p6 = jnp.pad(W2p, ((0, 0), (0, LOGIT_DIM - N_PED)))
    b1v_r = b1v.reshape(1, _H)
    b1p_r = b1p.reshape(1, _H)
    b2v_r = b2v.reshape(1, LOGIT_DIM)
    b2p_r = jnp.pad(b2p, (0, LOGIT_DIM - N_PED)).reshape(1, LOGIT_DIM)

    full = lambda i: (0, 0)
    lo = lambda i: (i, 0)
    hi = lambda i: (_HALFB + i, 0)
    half = _T // 2
    outs = pl.pallas_call(
        _body,
        grid=(_HALFB,),
        in_specs=[
            pl.BlockSpec((_BLK, _D), lo),
            pl.BlockSpec((_BLK, _D), hi),
            pl.BlockSpec((_BLK, 1), lo),
            pl.BlockSpec((_BLK, 1), hi),
            pl.BlockSpec((_D, _H), full),
            pl.BlockSpec((1, _H), full),
            pl.BlockSpec((_H, LOGIT_DIM), full),
            pl.BlockSpec((1, LOGIT_DIM), full),
            pl.BlockSpec((_D, _H), full),
            pl.BlockSpec((1, _H), full),
            pl.BlockSpec((_H, LOGIT_DIM), full),
            pl.BlockSpec((1, LOGIT_DIM), full),
        ],
        out_specs=[
            pl.BlockSpec((_BLK, LOGIT_DIM), lo),
            pl.BlockSpec((_BLK, LOGIT_DIM), hi),
            pl.BlockSpec((_BLK, 1), lo),
            pl.BlockSpec((_BLK, 1), hi),
            pl.BlockSpec((_BLK, 1), lo),
            pl.BlockSpec((_BLK, 1), hi),
        ],
        out_shape=[
            jax.ShapeDtypeStruct((_T, LOGIT_DIM), jnp.float32),
            jax.ShapeDtypeStruct((_T, LOGIT_DIM), jnp.float32),
            jax.ShapeDtypeStruct((_T, 1), jnp.bool_),
            jax.ShapeDtypeStruct((_T, 1), jnp.bool_),
            jax.ShapeDtypeStruct((_T, 1), jnp.bool_),
            jax.ShapeDtypeStruct((_T, 1), jnp.bool_),
        ],
    )(x, x, t, t, W1v, b1v_r, W2v, b2v_r, W1p, b1p_r, w2p6, b2p_r)
    outa, outb, mva, mvb, mpa, mpb = outs

    out = jnp.concatenate([outa[:half], outb[half:]])
    mv = jnp.concatenate([mva[:half], mvb[half:]])
    mp = jnp.concatenate([mpa[:half], mpb[half:]])

    return (out.reshape(_B, _N, LOGIT_DIM),
            mv.reshape(_B, _N),
            mp.reshape(_B, _N))
